# baseline (speedup 1.0000x reference)
import functools

import jax
import jax.numpy as jnp
from jax.experimental import pallas as pl
from jax.experimental.pallas import tpu as pltpu


def _interp_matrix(out_size: int, in_size: int) -> jnp.ndarray:
    scale = in_size / out_size
    pos = (jnp.arange(out_size, dtype=jnp.float32) + 0.5) * scale - 0.5
    i0 = jnp.floor(pos)
    frac = pos - i0
    i0c = jnp.clip(i0, 0, in_size - 1).astype(jnp.int32)
    i1c = jnp.clip(i0 + 1, 0, in_size - 1).astype(jnp.int32)
    u = ((1.0 - frac)[:, None] * jax.nn.one_hot(i0c, in_size, dtype=jnp.float32)
         + frac[:, None] * jax.nn.one_hot(i1c, in_size, dtype=jnp.float32))
    return u


def _mm(a, b):
    return jax.lax.dot_general(a, b, (((1,), (0,)), ((), ())),
                               preferred_element_type=jnp.float32)


def _mm_nt(a, b):
    return jax.lax.dot_general(a, b, (((1,), (1,)), ((), ())),
                               preferred_element_type=jnp.float32)


def _conv1_body(p_ref, w_ref, b_ref, o_ref):
    r = jnp.maximum(_mm(p_ref[...], w_ref[...]) + b_ref[...], 0.0)
    o_ref[...] = r.astype(jnp.bfloat16)


def _conv2_body(p_ref, w_ref, b_ref, o_ref):
    r = jnp.maximum(_mm(p_ref[...], w_ref[...]) + b_ref[...], 0.0)
    o_ref[...] = r.astype(jnp.bfloat16)


def _upsample_body(k_ref, c_ref, o_ref):
    o_ref[0] = _mm(k_ref[...], c_ref[0]).astype(jnp.bfloat16)


def _pack_body(q2_ref, q3_ref, o_ref):
    o_ref[:128, :] = q2_ref[...].T
    o_ref[128:, :] = q3_ref[...].T


def _knn_body(qt_ref, b_ref, o_ref, *, nq, cq):
    j = pl.program_id(0)

    @pl.when(j == 0)
    def _():
        o_ref[...] = jnp.full(o_ref.shape, -1e30, jnp.float32)

    bf = b_ref[...]
    b16 = (bf + bf).astype(jnp.bfloat16)
    bnorm = jnp.sum(bf * bf, axis=1, keepdims=True)

    for c in range(nq):
        sl = pl.ds(c * cq, cq)
        t2 = _mm(b16, qt_ref[:, sl])
        m = jnp.max(t2 - bnorm, axis=0, keepdims=True)
        o_ref[0, :, sl] = jnp.maximum(o_ref[0, :, sl], m)


def _finalize_body(m_ref, qt_ref, o_ref):
    ones = jnp.ones((1, 384), jnp.float32)
    qf = qt_ref[...].astype(jnp.float32)
    qsq = _mm(ones, qf * qf)
    o_ref[0] = jnp.sqrt(jnp.maximum(qsq - m_ref[0], 0.0))


def _resize_max_body(a_ref, l_ref, o_ref, s_ref):
    a = a_ref[0]
    l = l_ref[...]
    t = _mm(l, a)
    o = _mm_nt(t, l)
    o_ref[0, 0] = o
    s_ref[0] = jnp.max(o, axis=(0, 1), keepdims=True)


def kernel(x, W1, b1, W2, b2, memory_bank):
    B = x.shape[0]
    x16 = x.astype(jnp.bfloat16)
    xa = x16.reshape(B, 3, 28, 1792).transpose(0, 2, 1, 3)
    xa = jax.lax.optimization_barrier(xa)
    xb = xa.reshape(B, 28, 24, 28, 8).transpose(0, 1, 3, 2, 4)
    xb = jax.lax.optimization_barrier(xb)
    P1 = xb.reshape(B * 784, 192)
    W1m = W1.reshape(128, 192).T.astype(jnp.bfloat16)
    C1h = pl.pallas_call(
        _conv1_body,
        out_shape=jax.ShapeDtypeStruct((B * 784, 128), jnp.bfloat16),
    )(P1, W1m, b1.reshape(1, 128))

    P2 = (C1h.reshape(B, 14, 2, 14, 2, 128)
             .transpose(0, 1, 3, 2, 4, 5)
             .reshape(B * 196, 512))
    W2m = (W2.transpose(2, 3, 1, 0).reshape(512, 256)
             .astype(jnp.bfloat16))
    C2 = pl.pallas_call(
        _conv2_body,
        out_shape=jax.ShapeDtypeStruct((B * 196, 256), jnp.bfloat16),
    )(P2, W2m, b2.reshape(1, 256))

    U28 = _interp_matrix(28, 14)
    K = jnp.kron(U28, U28).astype(jnp.bfloat16)
    F3U = pl.pallas_call(
        _upsample_body,
        grid=(B,),
        in_specs=[
            pl.BlockSpec((784, 196), lambda b: (0, 0)),
            pl.BlockSpec((1, 196, 256), lambda b: (b, 0, 0)),
        ],
        out_specs=pl.BlockSpec((1, 784, 256), lambda b: (b, 0, 0)),
        out_shape=jax.ShapeDtypeStruct((B, 784, 256), jnp.bfloat16),
    )(K, C2.reshape(B, 196, 256)).reshape(B * 784, 256)

    NQTOT = B * 784
    TB, CQ = 2048, 1568
    NB, NQC = memory_bank.shape[0] // TB, NQTOT // CQ
    PQ = 896
    QT = pl.pallas_call(
        _pack_body,
        grid=(NQTOT // PQ,),
        in_specs=[
            pl.BlockSpec((PQ, 128), lambda c: (c, 0)),
            pl.BlockSpec((PQ, 256), lambda c: (c, 0)),
        ],
        out_specs=pl.BlockSpec((384, PQ), lambda c: (0, c)),
        out_shape=jax.ShapeDtypeStruct((384, NQTOT), jnp.bfloat16),
    )(C1h, F3U)
    rawm = pl.pallas_call(
        functools.partial(_knn_body, nq=NQC, cq=CQ),
        grid=(NB,),
        in_specs=[
            pl.BlockSpec((384, NQTOT), lambda j: (0, 0)),
            pl.BlockSpec((TB, 384), lambda j: (j, 0)),
        ],
        out_specs=pl.BlockSpec((1, 1, NQTOT), lambda j: (0, 0, 0)),
        out_shape=jax.ShapeDtypeStruct((1, 1, NQTOT), jnp.float32),
    )(QT, memory_bank)
    minds = pl.pallas_call(
        _finalize_body,
        out_shape=jax.ShapeDtypeStruct((1, 1, NQTOT), jnp.float32),
    )(rawm, QT)
    amap28 = minds.reshape(B, 28, 28)

    L224 = _interp_matrix(224, 28)
    anomaly_map, score = pl.pallas_call(
        _resize_max_body,
        grid=(B,),
        in_specs=[
            pl.BlockSpec((1, 28, 28), lambda b: (b, 0, 0)),
            pl.BlockSpec((224, 28), lambda b: (0, 0)),
        ],
        out_specs=[
            pl.BlockSpec((1, 1, 224, 224), lambda b: (b, 0, 0, 0)),
            pl.BlockSpec((1, 1, 1), lambda b: (b, 0, 0)),
        ],
        out_shape=[
            jax.ShapeDtypeStruct((B, 1, 224, 224), jnp.float32),
            jax.ShapeDtypeStruct((B, 1, 1), jnp.float32),
        ],
    )(amap28, L224)
    return (anomaly_map, score.reshape(B))


# speedup vs baseline: 1.0231x; 1.0231x over previous
import functools

import jax
import jax.numpy as jnp
from jax.experimental import pallas as pl
from jax.experimental.pallas import tpu as pltpu


def _interp_matrix(out_size: int, in_size: int) -> jnp.ndarray:
    scale = in_size / out_size
    pos = (jnp.arange(out_size, dtype=jnp.float32) + 0.5) * scale - 0.5
    i0 = jnp.floor(pos)
    frac = pos - i0
    i0c = jnp.clip(i0, 0, in_size - 1).astype(jnp.int32)
    i1c = jnp.clip(i0 + 1, 0, in_size - 1).astype(jnp.int32)
    u = ((1.0 - frac)[:, None] * jax.nn.one_hot(i0c, in_size, dtype=jnp.float32)
         + frac[:, None] * jax.nn.one_hot(i1c, in_size, dtype=jnp.float32))
    return u


def _mm(a, b):
    return jax.lax.dot_general(a, b, (((1,), (0,)), ((), ())),
                               preferred_element_type=jnp.float32)


def _mm_nt(a, b):
    return jax.lax.dot_general(a, b, (((1,), (1,)), ((), ())),
                               preferred_element_type=jnp.float32)


def _conv1_body(p_ref, w_ref, b_ref, o_ref):
    r = jnp.maximum(_mm(p_ref[...], w_ref[...]) + b_ref[...], 0.0)
    o_ref[...] = r.astype(jnp.bfloat16)


def _conv2_body(p_ref, w_ref, b_ref, o_ref):
    r = jnp.maximum(_mm(p_ref[...], w_ref[...]) + b_ref[...], 0.0)
    o_ref[...] = r.astype(jnp.bfloat16)


def _upsample_body(k_ref, c_ref, o_ref):
    o_ref[0] = _mm(k_ref[...], c_ref[0]).astype(jnp.bfloat16)


def _pack_body(q2_ref, q3_ref, o_ref):
    o_ref[:128, :] = q2_ref[...].T
    o_ref[128:, :] = q3_ref[...].T


def _knn_body(qt_ref, b_ref, o_ref, *, nq, cq):
    j = pl.program_id(0)

    @pl.when(j == 0)
    def _():
        o_ref[...] = jnp.full(o_ref.shape, -1e30, jnp.float32)

    bf = b_ref[...]
    b16 = (bf + bf).astype(jnp.bfloat16)
    bnorm = jnp.sum(bf * bf, axis=1, keepdims=True)

    for c in range(nq):
        sl = pl.ds(c * cq, cq)
        t2 = _mm(b16, qt_ref[:, sl])
        m = jnp.max(t2 - bnorm, axis=0, keepdims=True)
        o_ref[0, :, sl] = jnp.maximum(o_ref[0, :, sl], m)


def _finalize_body(m_ref, qt_ref, o_ref):
    ones = jnp.ones((1, 384), jnp.float32)
    qf = qt_ref[...].astype(jnp.float32)
    qsq = _mm(ones, qf * qf)
    o_ref[0] = jnp.sqrt(jnp.maximum(qsq - m_ref[0], 0.0))


def _resize_max_body(a_ref, l_ref, o_ref, s_ref):
    a = a_ref[0]
    l = l_ref[...]
    t = _mm(l, a)
    o = _mm_nt(t, l)
    o_ref[0, 0] = o
    s_ref[0] = jnp.max(o, axis=(0, 1), keepdims=True)


def kernel(x, W1, b1, W2, b2, memory_bank):
    B = x.shape[0]
    x16 = x.astype(jnp.bfloat16)
    xa = x16.reshape(B, 3, 28, 1792).transpose(0, 2, 1, 3)
    xa = jax.lax.optimization_barrier(xa)
    xb = xa.reshape(B, 28, 24, 28, 8).transpose(0, 1, 3, 2, 4)
    xb = jax.lax.optimization_barrier(xb)
    P1 = xb.reshape(B * 784, 192)
    W1m = W1.reshape(128, 192).T.astype(jnp.bfloat16)
    C1h = pl.pallas_call(
        _conv1_body,
        out_shape=jax.ShapeDtypeStruct((B * 784, 128), jnp.bfloat16),
    )(P1, W1m, b1.reshape(1, 128))

    P2 = (C1h.reshape(B, 14, 2, 14, 2, 128)
             .transpose(0, 1, 3, 2, 4, 5)
             .reshape(B * 196, 512))
    W2m = (W2.transpose(2, 3, 1, 0).reshape(512, 256)
             .astype(jnp.bfloat16))
    C2 = pl.pallas_call(
        _conv2_body,
        out_shape=jax.ShapeDtypeStruct((B * 196, 256), jnp.bfloat16),
    )(P2, W2m, b2.reshape(1, 256))

    U28 = _interp_matrix(28, 14)
    K = jnp.kron(U28, U28).astype(jnp.bfloat16)
    F3U = pl.pallas_call(
        _upsample_body,
        grid=(B,),
        in_specs=[
            pl.BlockSpec((784, 196), lambda b: (0, 0)),
            pl.BlockSpec((1, 196, 256), lambda b: (b, 0, 0)),
        ],
        out_specs=pl.BlockSpec((1, 784, 256), lambda b: (b, 0, 0)),
        out_shape=jax.ShapeDtypeStruct((B, 784, 256), jnp.bfloat16),
    )(K, C2.reshape(B, 196, 256)).reshape(B * 784, 256)

    NQTOT = B * 784
    TB, CQ = 2048, 896
    NB, NQC = memory_bank.shape[0] // TB, NQTOT // CQ
    PQ = 896
    QT = pl.pallas_call(
        _pack_body,
        grid=(NQTOT // PQ,),
        in_specs=[
            pl.BlockSpec((PQ, 128), lambda c: (c, 0)),
            pl.BlockSpec((PQ, 256), lambda c: (c, 0)),
        ],
        out_specs=pl.BlockSpec((384, PQ), lambda c: (0, c)),
        out_shape=jax.ShapeDtypeStruct((384, NQTOT), jnp.bfloat16),
    )(C1h, F3U)
    rawm = pl.pallas_call(
        functools.partial(_knn_body, nq=NQC, cq=CQ),
        grid=(NB,),
        in_specs=[
            pl.BlockSpec((384, NQTOT), lambda j: (0, 0)),
            pl.BlockSpec((TB, 384), lambda j: (j, 0)),
        ],
        out_specs=pl.BlockSpec((1, 1, NQTOT), lambda j: (0, 0, 0)),
        out_shape=jax.ShapeDtypeStruct((1, 1, NQTOT), jnp.float32),
    )(QT, memory_bank)
    minds = pl.pallas_call(
        _finalize_body,
        out_shape=jax.ShapeDtypeStruct((1, 1, NQTOT), jnp.float32),
    )(rawm, QT)
    amap28 = minds.reshape(B, 28, 28)

    L224 = _interp_matrix(224, 28)
    anomaly_map, score = pl.pallas_call(
        _resize_max_body,
        grid=(B,),
        in_specs=[
            pl.BlockSpec((1, 28, 28), lambda b: (b, 0, 0)),
            pl.BlockSpec((224, 28), lambda b: (0, 0)),
        ],
        out_specs=[
            pl.BlockSpec((1, 1, 224, 224), lambda b: (b, 0, 0, 0)),
            pl.BlockSpec((1, 1, 1), lambda b: (b, 0, 0)),
        ],
        out_shape=[
            jax.ShapeDtypeStruct((B, 1, 224, 224), jnp.float32),
            jax.ShapeDtypeStruct((B, 1, 1), jnp.float32),
        ],
    )(amap28, L224)
    return (anomaly_map, score.reshape(B))


# speedup vs baseline: 1.0673x; 1.0432x over previous
import functools

import jax
import jax.numpy as jnp
from jax.experimental import pallas as pl
from jax.experimental.pallas import tpu as pltpu


def _interp_matrix(out_size: int, in_size: int) -> jnp.ndarray:
    scale = in_size / out_size
    pos = (jnp.arange(out_size, dtype=jnp.float32) + 0.5) * scale - 0.5
    i0 = jnp.floor(pos)
    frac = pos - i0
    i0c = jnp.clip(i0, 0, in_size - 1).astype(jnp.int32)
    i1c = jnp.clip(i0 + 1, 0, in_size - 1).astype(jnp.int32)
    u = ((1.0 - frac)[:, None] * jax.nn.one_hot(i0c, in_size, dtype=jnp.float32)
         + frac[:, None] * jax.nn.one_hot(i1c, in_size, dtype=jnp.float32))
    return u


def _mm(a, b):
    return jax.lax.dot_general(a, b, (((1,), (0,)), ((), ())),
                               preferred_element_type=jnp.float32)


def _mm_nt(a, b):
    return jax.lax.dot_general(a, b, (((1,), (1,)), ((), ())),
                               preferred_element_type=jnp.float32)


def _conv1_body(p_ref, w_ref, b_ref, o_ref):
    r = jnp.maximum(_mm(p_ref[...], w_ref[...]) + b_ref[...], 0.0)
    o_ref[...] = r.astype(jnp.bfloat16)


def _conv2_body(p_ref, w_ref, b_ref, o_ref):
    r = jnp.maximum(_mm(p_ref[...], w_ref[...]) + b_ref[...], 0.0)
    o_ref[...] = r.astype(jnp.bfloat16)


def _upsample_body(k_ref, c_ref, o_ref):
    o_ref[0] = _mm(k_ref[...], c_ref[0]).astype(jnp.bfloat16)


def _knn_body(q2_ref, q3_ref, b_ref, o_ref, qt_ref, *, nb, nq, cq):
    j = pl.program_id(0)

    @pl.when(j == 0)
    def _():
        o_ref[...] = jnp.full(o_ref.shape, -1e30, jnp.float32)
        for c in range(nq):
            sl = pl.ds(c * cq, cq)
            qt_ref[:128, sl] = q2_ref[sl, :].T
            qt_ref[128:, sl] = q3_ref[sl, :].T

    bf = b_ref[...]
    b16 = (bf + bf).astype(jnp.bfloat16)
    bnorm = jnp.sum(bf * bf, axis=1, keepdims=True)

    for c in range(nq):
        sl = pl.ds(c * cq, cq)
        t2 = _mm(b16, qt_ref[:, sl])
        m = jnp.max(t2 - bnorm, axis=0, keepdims=True)
        o_ref[0, :, sl] = jnp.maximum(o_ref[0, :, sl], m)

    @pl.when(j == nb - 1)
    def _():
        ones = jnp.ones((1, 384), jnp.float32)
        for c in range(nq):
            sl = pl.ds(c * cq, cq)
            qf = qt_ref[:, sl].astype(jnp.float32)
            qsq = _mm(ones, qf * qf)
            o_ref[0, :, sl] = jnp.sqrt(
                jnp.maximum(qsq - o_ref[0, :, sl], 0.0))


def _resize_max_body(a_ref, l_ref, o_ref, s_ref):
    a = a_ref[0]
    l = l_ref[...]
    t = _mm(l, a)
    o = _mm_nt(t, l)
    o_ref[0, 0] = o
    s_ref[0] = jnp.max(o, axis=(0, 1), keepdims=True)


def kernel(x, W1, b1, W2, b2, memory_bank):
    B = x.shape[0]
    x16 = x.astype(jnp.bfloat16)
    xa = x16.reshape(B, 3, 28, 1792).transpose(0, 2, 1, 3)
    xa = jax.lax.optimization_barrier(xa)
    xb = xa.reshape(B, 28, 24, 28, 8).transpose(0, 1, 3, 2, 4)
    xb = jax.lax.optimization_barrier(xb)
    P1 = xb.reshape(B * 784, 192)
    W1m = W1.reshape(128, 192).T.astype(jnp.bfloat16)
    C1h = pl.pallas_call(
        _conv1_body,
        out_shape=jax.ShapeDtypeStruct((B * 784, 128), jnp.bfloat16),
    )(P1, W1m, b1.reshape(1, 128))

    P2 = (C1h.reshape(B, 14, 2, 14, 2, 128)
             .transpose(0, 1, 3, 2, 4, 5)
             .reshape(B * 196, 512))
    W2m = (W2.transpose(2, 3, 1, 0).reshape(512, 256)
             .astype(jnp.bfloat16))
    C2 = pl.pallas_call(
        _conv2_body,
        out_shape=jax.ShapeDtypeStruct((B * 196, 256), jnp.bfloat16),
    )(P2, W2m, b2.reshape(1, 256))

    U28 = _interp_matrix(28, 14)
    K = jnp.kron(U28, U28).astype(jnp.bfloat16)
    F3U = pl.pallas_call(
        _upsample_body,
        grid=(B,),
        in_specs=[
            pl.BlockSpec((784, 196), lambda b: (0, 0)),
            pl.BlockSpec((1, 196, 256), lambda b: (b, 0, 0)),
        ],
        out_specs=pl.BlockSpec((1, 784, 256), lambda b: (b, 0, 0)),
        out_shape=jax.ShapeDtypeStruct((B, 784, 256), jnp.bfloat16),
    )(K, C2.reshape(B, 196, 256)).reshape(B * 784, 256)

    NQTOT = B * 784
    TB, CQ = 2048, 896
    NB, NQC = memory_bank.shape[0] // TB, NQTOT // CQ
    minds = pl.pallas_call(
        functools.partial(_knn_body, nb=NB, nq=NQC, cq=CQ),
        grid=(NB,),
        in_specs=[
            pl.BlockSpec((NQTOT, 128), lambda j: (0, 0)),
            pl.BlockSpec((NQTOT, 256), lambda j: (0, 0)),
            pl.BlockSpec((TB, 384), lambda j: (j, 0)),
        ],
        out_specs=pl.BlockSpec((1, 1, NQTOT), lambda j: (0, 0, 0)),
        out_shape=jax.ShapeDtypeStruct((1, 1, NQTOT), jnp.float32),
        scratch_shapes=[pltpu.VMEM((384, NQTOT), jnp.bfloat16)],
    )(C1h, F3U, memory_bank)
    amap28 = minds.reshape(B, 28, 28)

    L224 = _interp_matrix(224, 28)
    anomaly_map, score = pl.pallas_call(
        _resize_max_body,
        grid=(B,),
        in_specs=[
            pl.BlockSpec((1, 28, 28), lambda b: (b, 0, 0)),
            pl.BlockSpec((224, 28), lambda b: (0, 0)),
        ],
        out_specs=[
            pl.BlockSpec((1, 1, 224, 224), lambda b: (b, 0, 0, 0)),
            pl.BlockSpec((1, 1, 1), lambda b: (b, 0, 0)),
        ],
        out_shape=[
            jax.ShapeDtypeStruct((B, 1, 224, 224), jnp.float32),
            jax.ShapeDtypeStruct((B, 1, 1), jnp.float32),
        ],
    )(amap28, L224)
    return (anomaly_map, score.reshape(B))
